# baseline (speedup 1.0000x reference)
import functools

import jax
import jax.numpy as jnp
from jax import lax
from jax.experimental import pallas as pl
from jax.experimental.pallas import tpu as pltpu
from jax.experimental.pallas import tpu_sc as plsc

N = 10000
E = 320000
D = 128
DO = 64
DH = D // 2
DOH = DO // 2

NC = 2
NS = 16
NW = NC * NS

CH = 128
RW = 80
EPAD = NW * RW * CH
EROWS = EPAD // CH
EWT = EPAD // NS
BE = 512
GP = EWT // BE
NPAD = 10240
TPT = NPAD // NS
TRASH = NPAD - 1

_mesh = plsc.VectorSubcoreMesh(
    core_axis_name="c", subcore_axis_name="s", num_cores=NC, num_subcores=NS
)


NPC2 = NPAD // NC
DSH = 5248
DTRASH = 5200


@functools.partial(
    pl.kernel,
    out_type=jax.ShapeDtypeStruct((NPAD, 8), jnp.float32),
    mesh=_mesh,
    scratch_types=[
        pltpu.VMEM((EWT,), jnp.int32),
        pltpu.VMEM((EWT,), jnp.int32),
        pltpu.VMEM((BE, 8), jnp.float32),
        pltpu.MemorySpace.VMEM_SHARED((DSH, 8), jnp.float32),
    ],
    compiler_params=pltpu.CompilerParams(use_tc_tiling_on_sc=False),
)
def _deg_kernel(dstp_hbm, ones_hbm, zrows_hbm, out_hbm, dst_v, dst2_v, ones_v,
                deg_sh):
    cid = lax.axis_index("c")
    sid = lax.axis_index("s")
    pltpu.sync_copy(dstp_hbm.at[pl.ds(sid * EWT, EWT)], dst_v)
    pltpu.sync_copy(ones_hbm, ones_v)
    pltpu.sync_copy(zrows_hbm, deg_sh.at[pl.ds(sid * (DSH // NS), DSH // NS)])
    base = cid * NPC2

    def rw(t, carry):
        v = dst_v[pl.ds(t * 16, 16)]
        loc = v - base
        inb = (loc >= 0) & (loc < NPC2)
        dst2_v[pl.ds(t * 16, 16)] = jnp.where(inb, loc, DTRASH)
        return carry

    lax.fori_loop(0, EWT // 16, rw, 0)
    plsc.subcore_barrier()

    def body(g, carry):
        pltpu.sync_copy(ones_v, deg_sh.at[dst2_v.at[pl.ds(g * BE, BE)]],
                        add=True)
        return carry

    lax.fori_loop(0, GP, body, 0)
    plsc.subcore_barrier()
    pltpu.sync_copy(
        deg_sh.at[pl.ds(sid * (NPC2 // NS), NPC2 // NS)],
        out_hbm.at[pl.ds(base + sid * (NPC2 // NS), NPC2 // NS)],
    )


def _make_spmm(wc):
    @functools.partial(
        pl.kernel,
        out_type=jax.ShapeDtypeStruct((NC, NPAD, wc), jnp.float32),
        mesh=_mesh,
        scratch_types=[
            pltpu.VMEM((EWT,), jnp.int32),
            pltpu.VMEM((EWT,), jnp.int32),
            pltpu.VMEM((BE, wc), jnp.float32),
            pltpu.MemorySpace.VMEM_SHARED((NPAD, wc), jnp.float32),
        ],
        compiler_params=pltpu.CompilerParams(use_tc_tiling_on_sc=False),
    )
    def spmm(table_hbm, srcp_hbm, dstp_hbm, zrows_hbm, out_hbm,
             src_v, dst_v, buf, agg_sh):
        cid = lax.axis_index("c")
        sid = lax.axis_index("s")
        pltpu.sync_copy(srcp_hbm.at[pl.ds(sid * EWT, EWT)], src_v)
        pltpu.sync_copy(dstp_hbm.at[pl.ds(sid * EWT, EWT)], dst_v)
        pltpu.sync_copy(zrows_hbm, agg_sh.at[pl.ds(sid * TPT, TPT)])
        plsc.subcore_barrier()
        tab = table_hbm.at[cid]

        def body(g, carry):
            pltpu.sync_copy(tab.at[src_v.at[pl.ds(g * BE, BE)]], buf)
            pltpu.sync_copy(buf, agg_sh.at[dst_v.at[pl.ds(g * BE, BE)]],
                            add=True)
            return carry

        lax.fori_loop(0, GP, body, 0)
        plsc.subcore_barrier()
        pltpu.sync_copy(
            agg_sh.at[pl.ds(sid * TPT, TPT)], out_hbm.at[cid, pl.ds(sid * TPT, TPT)]
        )

    return spmm


_spmm_h = _make_spmm(DH)
_spmm_q = _make_spmm(DOH)


BR = 2048
GR = NPAD // BR


def _rowspec(width):
    return pl.BlockSpec((BR, width), lambda i: (i, 0))


def _halfspec(width):
    return pl.BlockSpec((NC, BR, width), lambda i: (0, i, 0))


def _fullspec(shape):
    return pl.BlockSpec(shape, lambda i: (0,) * len(shape))


def _tc_pre_body(x_ref, d_ref, y_ref, n_ref):
    d = d_ref[...]
    nrm = jnp.where(d > 0, lax.rsqrt(jnp.maximum(d, 1.0)), 0.0)
    y = x_ref[...] * nrm
    y_ref[...] = jnp.stack([y[:, :DH], y[:, DH:]])
    n_ref[...] = nrm


_tc_pre = pl.pallas_call(
    _tc_pre_body,
    grid=(GR,),
    in_specs=[_rowspec(D), _rowspec(1)],
    out_specs=[_halfspec(DH), _rowspec(1)],
    out_shape=[
        jax.ShapeDtypeStruct((NC, NPAD, DH), jnp.float32),
        jax.ShapeDtypeStruct((NPAD, 1), jnp.float32),
    ],
)


def _tc_mid1_body(a_ref, n_ref, w_ref, b_ref, y_ref):
    nrm = n_ref[...]
    agg = jnp.concatenate([a_ref[0], a_ref[1]], axis=1) * nrm
    h = jnp.dot(agg, w_ref[...], preferred_element_type=jnp.float32) + b_ref[...]
    y = h * nrm
    y_ref[...] = jnp.stack([y[:, :DH], y[:, DH:]])


_tc_mid1 = pl.pallas_call(
    _tc_mid1_body,
    grid=(GR,),
    in_specs=[_halfspec(DH), _rowspec(1), _fullspec((D, D)), _fullspec((1, D))],
    out_specs=_halfspec(DH),
    out_shape=jax.ShapeDtypeStruct((NC, NPAD, DH), jnp.float32),
)


def _tc_mid2_body(a_ref, n_ref, w2_ref, b2_ref, w3_ref, y_ref):
    nrm = n_ref[...]
    agg = jnp.concatenate([a_ref[0], a_ref[1]], axis=1) * nrm
    h = jnp.dot(agg, w2_ref[...], preferred_element_type=jnp.float32) + b2_ref[...]
    y = jnp.dot(h * nrm, w3_ref[...], preferred_element_type=jnp.float32)
    y_ref[...] = jnp.stack([y[:, :DOH], y[:, DOH:]])


_tc_mid2 = pl.pallas_call(
    _tc_mid2_body,
    grid=(GR,),
    in_specs=[_halfspec(DH), _rowspec(1), _fullspec((D, D)), _fullspec((1, D)),
              _fullspec((D, DO))],
    out_specs=_halfspec(DOH),
    out_shape=jax.ShapeDtypeStruct((NC, NPAD, DOH), jnp.float32),
)


def _tc_post_body(a_ref, n_ref, b_ref, o_ref):
    h = jnp.concatenate([a_ref[0], a_ref[1]], axis=1) * n_ref[...] + b_ref[...]
    m = jnp.max(h, axis=1, keepdims=True)
    ex = jnp.exp(h - m)
    s = jnp.sum(ex, axis=1, keepdims=True)
    o_ref[...] = (h - m) - jnp.log(s)


_tc_post = pl.pallas_call(
    _tc_post_body,
    grid=(GR,),
    in_specs=[_halfspec(DOH), _rowspec(1), _fullspec((1, DO))],
    out_specs=_rowspec(DO),
    out_shape=jax.ShapeDtypeStruct((NPAD, DO), jnp.float32),
)


def kernel(features, edge_index, W1, b1, W2, b2, W3, b3):
    f32 = jnp.float32
    src = edge_index[0].astype(jnp.int32)
    dst = edge_index[1].astype(jnp.int32)
    srcp = jnp.concatenate([src, jnp.zeros((EPAD - E,), jnp.int32)])
    dstp = jnp.concatenate([dst, jnp.full((EPAD - E,), TRASH, jnp.int32)])
    xpad = jnp.pad(features.astype(f32), ((0, NPAD - N), (0, 0)))
    ones8 = jnp.ones((BE, 8), f32)
    z8 = jnp.zeros((DSH // NS, 8), f32)
    zh = jnp.zeros((TPT, DH), f32)
    zq = jnp.zeros((TPT, DOH), f32)

    deg8 = _deg_kernel(dstp, ones8, z8)
    y0, nrm = _tc_pre(xpad, deg8[:, 0:1])
    agg = _spmm_h(y0, srcp, dstp, zh)
    y1 = _tc_mid1(agg, nrm, W1, b1.reshape(1, D))
    agg = _spmm_h(y1, srcp, dstp, zh)
    y2 = _tc_mid2(agg, nrm, W2, b2.reshape(1, D), W3)
    agg = _spmm_q(y2, srcp, dstp, zq)
    out = _tc_post(agg, nrm, b3.reshape(1, DO))
    return out[:N]


# speedup vs baseline: 1.2996x; 1.2996x over previous
import functools

import jax
import jax.numpy as jnp
from jax import lax
from jax.experimental import pallas as pl
from jax.experimental.pallas import tpu as pltpu
from jax.experimental.pallas import tpu_sc as plsc

N = 10000
E = 320000
D = 128
DO = 64
DH = D // 2
DOH = DO // 2

NC = 2
NS = 16
NW = NC * NS

CH = 128
RW = 80
EPAD = NW * RW * CH
EROWS = EPAD // CH
EWT = EPAD // NS
BE = 640
GP = EWT // BE
NPAD = 10240
TPT = NPAD // NS
TRASH = NPAD - 1

_mesh = plsc.VectorSubcoreMesh(
    core_axis_name="c", subcore_axis_name="s", num_cores=NC, num_subcores=NS
)


@functools.partial(
    pl.kernel,
    out_type=jax.ShapeDtypeStruct((NC, NPAD, 8), jnp.float32),
    mesh=_mesh,
    scratch_types=[
        pltpu.VMEM((RW, CH), jnp.int32),
        pltpu.VMEM((CH, 8), jnp.float32),
        pltpu.MemorySpace.VMEM_SHARED((NPAD, 8), jnp.float32),
    ],
    compiler_params=pltpu.CompilerParams(use_tc_tiling_on_sc=False),
)
def _deg_kernel(dstp_hbm, ones_hbm, zrows_hbm, out_hbm, dst_v, ones_v, deg_sh):
    cid = lax.axis_index("c")
    sid = lax.axis_index("s")
    w = cid * NS + sid
    pltpu.sync_copy(dstp_hbm.at[pl.ds(w * RW, RW)], dst_v)
    pltpu.sync_copy(ones_hbm, ones_v)
    pltpu.sync_copy(zrows_hbm, deg_sh.at[pl.ds(sid * TPT, TPT)])
    plsc.subcore_barrier()

    def body(j, carry):
        pltpu.sync_copy(ones_v, deg_sh.at[dst_v.at[j]], add=True)
        return carry

    lax.fori_loop(0, RW, body, 0)
    plsc.subcore_barrier()
    pltpu.sync_copy(
        deg_sh.at[pl.ds(sid * TPT, TPT)], out_hbm.at[cid, pl.ds(sid * TPT, TPT)]
    )


def _make_spmm(wc):
    @functools.partial(
        pl.kernel,
        out_type=jax.ShapeDtypeStruct((NC, NPAD, wc), jnp.float32),
        mesh=_mesh,
        scratch_types=[
            pltpu.VMEM((EWT,), jnp.int32),
            pltpu.VMEM((EWT,), jnp.int32),
            pltpu.VMEM((BE, wc), jnp.float32),
            pltpu.MemorySpace.VMEM_SHARED((NPAD, wc), jnp.float32),
        ],
        compiler_params=pltpu.CompilerParams(use_tc_tiling_on_sc=False),
    )
    def spmm(table_hbm, srcp_hbm, dstp_hbm, zrows_hbm, out_hbm,
             src_v, dst_v, buf, agg_sh):
        cid = lax.axis_index("c")
        sid = lax.axis_index("s")
        pltpu.sync_copy(srcp_hbm.at[pl.ds(sid * EWT, EWT)], src_v)
        pltpu.sync_copy(dstp_hbm.at[pl.ds(sid * EWT, EWT)], dst_v)
        pltpu.sync_copy(zrows_hbm, agg_sh.at[pl.ds(sid * TPT, TPT)])
        plsc.subcore_barrier()
        tab = table_hbm.at[cid]

        def body(g, carry):
            pltpu.sync_copy(tab.at[src_v.at[pl.ds(g * BE, BE)]], buf)
            pltpu.sync_copy(buf, agg_sh.at[dst_v.at[pl.ds(g * BE, BE)]],
                            add=True)
            return carry

        lax.fori_loop(0, GP, body, 0)
        plsc.subcore_barrier()
        pltpu.sync_copy(
            agg_sh.at[pl.ds(sid * TPT, TPT)], out_hbm.at[cid, pl.ds(sid * TPT, TPT)]
        )

    return spmm


_spmm_h = _make_spmm(DH)
_spmm_q = _make_spmm(DOH)


BR = 2048
GR = NPAD // BR


def _rowspec(width):
    return pl.BlockSpec((BR, width), lambda i: (i, 0))


def _halfspec(width):
    return pl.BlockSpec((NC, BR, width), lambda i: (0, i, 0))


def _fullspec(shape):
    return pl.BlockSpec(shape, lambda i: (0,) * len(shape))


def _tc_pre_body(x_ref, d0_ref, d1_ref, y_ref, n_ref):
    d = d0_ref[...] + d1_ref[...]
    nrm = jnp.where(d > 0, lax.rsqrt(jnp.maximum(d, 1.0)), 0.0)
    y = x_ref[...] * nrm
    y_ref[...] = jnp.stack([y[:, :DH], y[:, DH:]])
    n_ref[...] = nrm


_tc_pre = pl.pallas_call(
    _tc_pre_body,
    grid=(GR,),
    in_specs=[_rowspec(D), _rowspec(1), _rowspec(1)],
    out_specs=[_halfspec(DH), _rowspec(1)],
    out_shape=[
        jax.ShapeDtypeStruct((NC, NPAD, DH), jnp.float32),
        jax.ShapeDtypeStruct((NPAD, 1), jnp.float32),
    ],
)


def _tc_mid1_body(a_ref, n_ref, w_ref, b_ref, y_ref):
    nrm = n_ref[...]
    agg = jnp.concatenate([a_ref[0], a_ref[1]], axis=1) * nrm
    h = jnp.dot(agg, w_ref[...], preferred_element_type=jnp.float32) + b_ref[...]
    y = h * nrm
    y_ref[...] = jnp.stack([y[:, :DH], y[:, DH:]])


_tc_mid1 = pl.pallas_call(
    _tc_mid1_body,
    grid=(GR,),
    in_specs=[_halfspec(DH), _rowspec(1), _fullspec((D, D)), _fullspec((1, D))],
    out_specs=_halfspec(DH),
    out_shape=jax.ShapeDtypeStruct((NC, NPAD, DH), jnp.float32),
)


def _tc_mid2_body(a_ref, n_ref, w2_ref, b2_ref, w3_ref, y_ref):
    nrm = n_ref[...]
    agg = jnp.concatenate([a_ref[0], a_ref[1]], axis=1) * nrm
    h = jnp.dot(agg, w2_ref[...], preferred_element_type=jnp.float32) + b2_ref[...]
    y = jnp.dot(h * nrm, w3_ref[...], preferred_element_type=jnp.float32)
    y_ref[...] = jnp.stack([y[:, :DOH], y[:, DOH:]])


_tc_mid2 = pl.pallas_call(
    _tc_mid2_body,
    grid=(GR,),
    in_specs=[_halfspec(DH), _rowspec(1), _fullspec((D, D)), _fullspec((1, D)),
              _fullspec((D, DO))],
    out_specs=_halfspec(DOH),
    out_shape=jax.ShapeDtypeStruct((NC, NPAD, DOH), jnp.float32),
)


def _tc_post_body(a_ref, n_ref, b_ref, o_ref):
    h = jnp.concatenate([a_ref[0], a_ref[1]], axis=1) * n_ref[...] + b_ref[...]
    m = jnp.max(h, axis=1, keepdims=True)
    ex = jnp.exp(h - m)
    s = jnp.sum(ex, axis=1, keepdims=True)
    o_ref[...] = (h - m) - jnp.log(s)


_tc_post = pl.pallas_call(
    _tc_post_body,
    grid=(GR,),
    in_specs=[_halfspec(DOH), _rowspec(1), _fullspec((1, DO))],
    out_specs=_rowspec(DO),
    out_shape=jax.ShapeDtypeStruct((NPAD, DO), jnp.float32),
)


def kernel(features, edge_index, W1, b1, W2, b2, W3, b3):
    f32 = jnp.float32
    src = edge_index[0].astype(jnp.int32)
    dst = edge_index[1].astype(jnp.int32)
    srcp = jnp.concatenate([src, jnp.zeros((EPAD - E,), jnp.int32)])
    dstp = jnp.concatenate([dst, jnp.full((EPAD - E,), TRASH, jnp.int32)])
    dstp_deg = dstp.reshape(EROWS, CH)
    xpad = jnp.pad(features.astype(f32), ((0, NPAD - N), (0, 0)))
    ones8 = jnp.ones((CH, 8), f32)
    z8 = jnp.zeros((TPT, 8), f32)
    zh = jnp.zeros((TPT, DH), f32)
    zq = jnp.zeros((TPT, DOH), f32)

    deg = _deg_kernel(dstp_deg, ones8, z8)
    d0 = deg[0, :, 0:1]
    d1 = deg[1, :, 0:1]
    y0, nrm = _tc_pre(xpad, d0, d1)
    agg = _spmm_h(y0, srcp, dstp, zh)
    y1 = _tc_mid1(agg, nrm, W1, b1.reshape(1, D))
    agg = _spmm_h(y1, srcp, dstp, zh)
    y2 = _tc_mid2(agg, nrm, W2, b2.reshape(1, D), W3)
    agg = _spmm_q(y2, srcp, dstp, zq)
    out = _tc_post(agg, nrm, b3.reshape(1, DO))
    return out[:N]


# speedup vs baseline: 1.3080x; 1.0065x over previous
import functools

import jax
import jax.numpy as jnp
from jax import lax
from jax.experimental import pallas as pl
from jax.experimental.pallas import tpu as pltpu
from jax.experimental.pallas import tpu_sc as plsc

N = 10000
E = 320000
D = 128
DO = 64
DH = D // 2
DOH = DO // 2

NC = 2
NS = 16
NW = NC * NS

CH = 128
RW = 80
EPAD = NW * RW * CH
EROWS = EPAD // CH
EWT = EPAD // NS
BE = 640
GP = EWT // BE
NPAD = 10240
TPT = NPAD // NS
TRASH = NPAD - 1

_mesh = plsc.VectorSubcoreMesh(
    core_axis_name="c", subcore_axis_name="s", num_cores=NC, num_subcores=NS
)


@functools.partial(
    pl.kernel,
    out_type=jax.ShapeDtypeStruct((NC, NPAD, 8), jnp.float32),
    mesh=_mesh,
    scratch_types=[
        pltpu.VMEM((RW, CH), jnp.int32),
        pltpu.VMEM((CH, 8), jnp.float32),
        pltpu.MemorySpace.VMEM_SHARED((NPAD, 8), jnp.float32),
    ],
    compiler_params=pltpu.CompilerParams(use_tc_tiling_on_sc=False),
)
def _deg_kernel(dstp_hbm, ones_hbm, zrows_hbm, out_hbm, dst_v, ones_v, deg_sh):
    cid = lax.axis_index("c")
    sid = lax.axis_index("s")
    w = cid * NS + sid
    pltpu.sync_copy(dstp_hbm.at[pl.ds(w * RW, RW)], dst_v)
    pltpu.sync_copy(ones_hbm, ones_v)
    pltpu.sync_copy(zrows_hbm, deg_sh.at[pl.ds(sid * TPT, TPT)])
    plsc.subcore_barrier()

    def body(j, carry):
        pltpu.sync_copy(ones_v, deg_sh.at[dst_v.at[j]], add=True)
        return carry

    lax.fori_loop(0, RW, body, 0)
    plsc.subcore_barrier()
    pltpu.sync_copy(
        deg_sh.at[pl.ds(sid * TPT, TPT)], out_hbm.at[cid, pl.ds(sid * TPT, TPT)]
    )


def _make_spmm(wc, be):
    gp = EWT // be

    @functools.partial(
        pl.kernel,
        out_type=jax.ShapeDtypeStruct((NC, NPAD, wc), jnp.float32),
        mesh=_mesh,
        scratch_types=[
            pltpu.VMEM((EWT,), jnp.int32),
            pltpu.VMEM((EWT,), jnp.int32),
            pltpu.VMEM((be, wc), jnp.float32),
            pltpu.MemorySpace.VMEM_SHARED((NPAD, wc), jnp.float32),
        ],
        compiler_params=pltpu.CompilerParams(use_tc_tiling_on_sc=False),
    )
    def spmm(table_hbm, srcp_hbm, dstp_hbm, zrows_hbm, out_hbm,
             src_v, dst_v, buf, agg_sh):
        cid = lax.axis_index("c")
        sid = lax.axis_index("s")
        pltpu.sync_copy(srcp_hbm.at[pl.ds(sid * EWT, EWT)], src_v)
        pltpu.sync_copy(dstp_hbm.at[pl.ds(sid * EWT, EWT)], dst_v)
        pltpu.sync_copy(zrows_hbm, agg_sh.at[pl.ds(sid * TPT, TPT)])
        plsc.subcore_barrier()
        tab = table_hbm.at[cid]

        def body(g, carry):
            pltpu.sync_copy(tab.at[src_v.at[pl.ds(g * be, be)]], buf)
            pltpu.sync_copy(buf, agg_sh.at[dst_v.at[pl.ds(g * be, be)]],
                            add=True)
            return carry

        lax.fori_loop(0, gp, body, 0)
        plsc.subcore_barrier()
        pltpu.sync_copy(
            agg_sh.at[pl.ds(sid * TPT, TPT)], out_hbm.at[cid, pl.ds(sid * TPT, TPT)]
        )

    return spmm


_spmm_h = _make_spmm(DH, BE)
_spmm_q = _make_spmm(DOH, 2 * BE)


BR = 2048
GR = NPAD // BR


def _rowspec(width):
    return pl.BlockSpec((BR, width), lambda i: (i, 0))


def _halfspec(width):
    return pl.BlockSpec((NC, BR, width), lambda i: (0, i, 0))


def _fullspec(shape):
    return pl.BlockSpec(shape, lambda i: (0,) * len(shape))


def _tc_pre_body(x_ref, d0_ref, d1_ref, y_ref, n_ref):
    d = d0_ref[...] + d1_ref[...]
    nrm = jnp.where(d > 0, lax.rsqrt(jnp.maximum(d, 1.0)), 0.0)
    y = x_ref[...] * nrm
    y_ref[...] = jnp.stack([y[:, :DH], y[:, DH:]])
    n_ref[...] = nrm


_tc_pre = pl.pallas_call(
    _tc_pre_body,
    grid=(GR,),
    in_specs=[_rowspec(D), _rowspec(1), _rowspec(1)],
    out_specs=[_halfspec(DH), _rowspec(1)],
    out_shape=[
        jax.ShapeDtypeStruct((NC, NPAD, DH), jnp.float32),
        jax.ShapeDtypeStruct((NPAD, 1), jnp.float32),
    ],
)


def _tc_mid1_body(a_ref, n_ref, w_ref, b_ref, y_ref):
    nrm = n_ref[...]
    agg = jnp.concatenate([a_ref[0], a_ref[1]], axis=1) * nrm
    h = jnp.dot(agg, w_ref[...], preferred_element_type=jnp.float32) + b_ref[...]
    y = h * nrm
    y_ref[...] = jnp.stack([y[:, :DH], y[:, DH:]])


_tc_mid1 = pl.pallas_call(
    _tc_mid1_body,
    grid=(GR,),
    in_specs=[_halfspec(DH), _rowspec(1), _fullspec((D, D)), _fullspec((1, D))],
    out_specs=_halfspec(DH),
    out_shape=jax.ShapeDtypeStruct((NC, NPAD, DH), jnp.float32),
)


def _tc_mid2_body(a_ref, n_ref, w2_ref, b2_ref, w3_ref, y_ref):
    nrm = n_ref[...]
    agg = jnp.concatenate([a_ref[0], a_ref[1]], axis=1) * nrm
    h = jnp.dot(agg, w2_ref[...], preferred_element_type=jnp.float32) + b2_ref[...]
    y = jnp.dot(h * nrm, w3_ref[...], preferred_element_type=jnp.float32)
    y_ref[...] = jnp.stack([y[:, :DOH], y[:, DOH:]])


_tc_mid2 = pl.pallas_call(
    _tc_mid2_body,
    grid=(GR,),
    in_specs=[_halfspec(DH), _rowspec(1), _fullspec((D, D)), _fullspec((1, D)),
              _fullspec((D, DO))],
    out_specs=_halfspec(DOH),
    out_shape=jax.ShapeDtypeStruct((NC, NPAD, DOH), jnp.float32),
)


def _tc_post_body(a_ref, n_ref, b_ref, o_ref):
    h = jnp.concatenate([a_ref[0], a_ref[1]], axis=1) * n_ref[...] + b_ref[...]
    m = jnp.max(h, axis=1, keepdims=True)
    ex = jnp.exp(h - m)
    s = jnp.sum(ex, axis=1, keepdims=True)
    o_ref[...] = (h - m) - jnp.log(s)


_tc_post = pl.pallas_call(
    _tc_post_body,
    grid=(GR,),
    in_specs=[_halfspec(DOH), _rowspec(1), _fullspec((1, DO))],
    out_specs=_rowspec(DO),
    out_shape=jax.ShapeDtypeStruct((NPAD, DO), jnp.float32),
)


def kernel(features, edge_index, W1, b1, W2, b2, W3, b3):
    f32 = jnp.float32
    src = edge_index[0].astype(jnp.int32)
    dst = edge_index[1].astype(jnp.int32)
    srcp = jnp.concatenate([src, jnp.zeros((EPAD - E,), jnp.int32)])
    dstp = jnp.concatenate([dst, jnp.full((EPAD - E,), TRASH, jnp.int32)])
    dstp_deg = dstp.reshape(EROWS, CH)
    xpad = jnp.pad(features.astype(f32), ((0, NPAD - N), (0, 0)))
    ones8 = jnp.ones((CH, 8), f32)
    z8 = jnp.zeros((TPT, 8), f32)
    zh = jnp.zeros((TPT, DH), f32)
    zq = jnp.zeros((TPT, DOH), f32)

    deg = _deg_kernel(dstp_deg, ones8, z8)
    d0 = deg[0, :, 0:1]
    d1 = deg[1, :, 0:1]
    y0, nrm = _tc_pre(xpad, d0, d1)
    agg = _spmm_h(y0, srcp, dstp, zh)
    y1 = _tc_mid1(agg, nrm, W1, b1.reshape(1, D))
    agg = _spmm_h(y1, srcp, dstp, zh)
    y2 = _tc_mid2(agg, nrm, W2, b2.reshape(1, D), W3)
    agg = _spmm_q(y2, srcp, dstp, zq)
    out = _tc_post(agg, nrm, b3.reshape(1, DO))
    return out[:N]
